# Initial kernel scaffold; baseline (speedup 1.0000x reference)
import jax
import jax.numpy as jnp
from jax.experimental import pallas as pl


def kernel(node_states, rel_0, rel_1, rel_2, W1_0, b1_0, W2_0, b2_0, W1_1, b1_1, W2_1, b2_1, W1_2, b1_2, W2_2, b2_2, U1, bU1, U2, bU2):
    raise NotImplementedError("write your pallas kernel here")


# speedup vs baseline: 2.1801x; 2.1801x over previous
import functools

import jax
import jax.numpy as jnp
from jax import lax
from jax.experimental import pallas as pl
from jax.experimental.pallas import tpu as pltpu
from jax.experimental.pallas import tpu_sc as plsc

H = 128
N = 10000
NC = 2
NS = 16
NW = NC * NS

CNTS = (300000, 200000, 60000)
CHUNKS = (120, 80, 120)


def _worker_id():
    return lax.axis_index("s") * NC + lax.axis_index("c")


def _gather_body(ns_hbm, idx0, idx1, idx2, out0, out1, out2,
                 ia_v, ra_v, ib_v, rb_v, sem):
    wid = _worker_id()

    def do_rel(idx_hbm, out_hbm, ch, cnt, idx_v, rows_v):
        nchunks = cnt // ch
        nt = (nchunks + NW - 1) // NW

        def body(t, carry):
            j = wid + NW * t

            @pl.when(j < nchunks)
            def _():
                base = j * ch
                pltpu.sync_copy(idx_hbm.at[pl.ds(base, ch)], idx_v)
                pltpu.async_copy(ns_hbm.at[idx_v], rows_v, sem).wait()
                pltpu.sync_copy(rows_v, out_hbm.at[pl.ds(base, ch)])
            return carry

        lax.fori_loop(0, nt, body, 0)

    do_rel(idx0, out0, CHUNKS[0], CNTS[0], ia_v, ra_v)
    do_rel(idx1, out1, CHUNKS[1], CNTS[1], ib_v, rb_v)
    do_rel(idx2, out2, CHUNKS[2], CNTS[2], ia_v, ra_v)


_gather_call = pl.kernel(
    _gather_body,
    out_type=[jax.ShapeDtypeStruct((c, H), jnp.float32) for c in CNTS],
    mesh=plsc.VectorSubcoreMesh(core_axis_name="c", subcore_axis_name="s",
                                num_cores=NC, num_subcores=NS),
    scratch_types=[
        pltpu.VMEM((CHUNKS[0],), jnp.int32),
        pltpu.VMEM((CHUNKS[0], H), jnp.float32),
        pltpu.VMEM((CHUNKS[1],), jnp.int32),
        pltpu.VMEM((CHUNKS[1], H), jnp.float32),
        pltpu.SemaphoreType.DMA,
    ],
)


_ZROWS = 624
_ZREM = N - _ZROWS * NS


def _scatter_body(e0, e1, e2, idx0, idx1, idx2, zeros_hbm, outa, outb,
                  acc_sh, ia_v, ra_v, ib_v, rb_v, sem):
    cid = lax.axis_index("c")
    sid = lax.axis_index("s")
    wid = _worker_id()

    row0 = sid * _ZROWS
    pltpu.sync_copy(zeros_hbm.at[pl.ds(row0, _ZROWS)],
                    acc_sh.at[pl.ds(row0, _ZROWS)])

    @pl.when(sid == NS - 1)
    def _():
        pltpu.sync_copy(zeros_hbm.at[pl.ds(_ZROWS * NS, _ZREM)],
                        acc_sh.at[pl.ds(_ZROWS * NS, _ZREM)])

    plsc.subcore_barrier()

    def do_rel(e_hbm, idx_hbm, ch, cnt, idx_v, rows_v):
        nchunks = cnt // ch
        nt = (nchunks + NW - 1) // NW

        def body(t, carry):
            j = wid + NW * t

            @pl.when(j < nchunks)
            def _():
                base = j * ch
                pltpu.sync_copy(idx_hbm.at[pl.ds(base, ch)], idx_v)
                pltpu.sync_copy(e_hbm.at[pl.ds(base, ch)], rows_v)
                pltpu.sync_copy(rows_v, acc_sh.at[idx_v], add=True)
            return carry

        lax.fori_loop(0, nt, body, 0)

    do_rel(e0, idx0, CHUNKS[0], CNTS[0], ia_v, ra_v)
    do_rel(e1, idx1, CHUNKS[1], CNTS[1], ib_v, rb_v)
    do_rel(e2, idx2, CHUNKS[2], CNTS[2], ia_v, ra_v)

    plsc.subcore_barrier()

    @pl.when(cid == 0)
    def _():
        pltpu.sync_copy(acc_sh.at[pl.ds(row0, _ZROWS)],
                        outa.at[pl.ds(row0, _ZROWS)])

        @pl.when(sid == NS - 1)
        def _():
            pltpu.sync_copy(acc_sh.at[pl.ds(_ZROWS * NS, _ZREM)],
                            outa.at[pl.ds(_ZROWS * NS, _ZREM)])

    @pl.when(cid == 1)
    def _():
        pltpu.sync_copy(acc_sh.at[pl.ds(row0, _ZROWS)],
                        outb.at[pl.ds(row0, _ZROWS)])

        @pl.when(sid == NS - 1)
        def _():
            pltpu.sync_copy(acc_sh.at[pl.ds(_ZROWS * NS, _ZREM)],
                            outb.at[pl.ds(_ZROWS * NS, _ZREM)])


_scatter_call = pl.kernel(
    _scatter_body,
    out_type=[jax.ShapeDtypeStruct((N, H), jnp.float32),
              jax.ShapeDtypeStruct((N, H), jnp.float32)],
    mesh=plsc.VectorSubcoreMesh(core_axis_name="c", subcore_axis_name="s",
                                num_cores=NC, num_subcores=NS),
    scratch_types=[
        pltpu.VMEM_SHARED((N, H), jnp.float32),
        pltpu.VMEM((CHUNKS[0],), jnp.int32),
        pltpu.VMEM((CHUNKS[0], H), jnp.float32),
        pltpu.VMEM((CHUNKS[1],), jnp.int32),
        pltpu.VMEM((CHUNKS[1], H), jnp.float32),
        pltpu.SemaphoreType.DMA,
    ],
)


def _mlp_body(x_ref, w1_ref, b1_ref, w2_ref, b2_ref, o_ref, mx_ref):
    x = x_ref[...]
    h = lax.dot_general(x, w1_ref[...], (((1,), (1,)), ((), ())),
                        preferred_element_type=jnp.float32) + b1_ref[...]
    h = jnp.maximum(h, 0.0)
    o = lax.dot_general(h, w2_ref[...], (((1,), (1,)), ((), ())),
                        preferred_element_type=jnp.float32) + b2_ref[...]
    o_ref[...] = o
    bm = jnp.full((1, 1), jnp.max(o), jnp.float32)
    i = pl.program_id(0)

    @pl.when(i == 0)
    def _():
        mx_ref[...] = bm

    @pl.when(i > 0)
    def _():
        mx_ref[...] = jnp.maximum(mx_ref[...], bm)


def _relation_mlp(x, w1, b1, w2, b2, bt):
    t, io = x.shape
    o, mx = pl.pallas_call(
        _mlp_body,
        grid=(t // bt,),
        in_specs=[
            pl.BlockSpec((bt, io), lambda i: (i, 0)),
            pl.BlockSpec((io, io), lambda i: (0, 0)),
            pl.BlockSpec((1, io), lambda i: (0, 0)),
            pl.BlockSpec((io, io), lambda i: (0, 0)),
            pl.BlockSpec((1, io), lambda i: (0, 0)),
        ],
        out_specs=[
            pl.BlockSpec((bt, io), lambda i: (i, 0)),
            pl.BlockSpec((1, 1), lambda i: (0, 0)),
        ],
        out_shape=[
            jax.ShapeDtypeStruct((t, io), jnp.float32),
            jax.ShapeDtypeStruct((1, 1), jnp.float32),
        ],
    )(x, w1, b1.reshape(1, io), w2, b2.reshape(1, io))
    return o, mx[0, 0]


def _exp_body(o_ref, m_ref, e_ref):
    e_ref[...] = jnp.exp(8.0 * (o_ref[...] - m_ref[0]))


def _exp_kernel(o, mx, bt):
    t, io = o.shape
    return pl.pallas_call(
        _exp_body,
        grid=(t // bt,),
        in_specs=[
            pl.BlockSpec((bt, io), lambda i: (i, 0)),
            pl.BlockSpec(memory_space=pltpu.SMEM),
        ],
        out_specs=pl.BlockSpec((bt, io), lambda i: (i, 0)),
        out_shape=jax.ShapeDtypeStruct((t, io), jnp.float32),
    )(o, mx.reshape(1))


def _final_body(pa_ref, pb_ref, ns_ref, u1a_ref, u1b_ref, bu1_ref, u2_ref,
                bu2_ref, m_ref, out_ref):
    p = pa_ref[...] + pb_ref[...] + 1e-16
    mm = jnp.log(p) * 0.125 + m_ref[0]
    h = (lax.dot_general(mm, u1a_ref[...], (((1,), (1,)), ((), ())),
                         preferred_element_type=jnp.float32)
         + lax.dot_general(ns_ref[...], u1b_ref[...], (((1,), (1,)), ((), ())),
                           preferred_element_type=jnp.float32)
         + bu1_ref[...])
    h = jnp.maximum(h, 0.0)
    out_ref[...] = lax.dot_general(h, u2_ref[...], (((1,), (1,)), ((), ())),
                                   preferred_element_type=jnp.float32) + bu2_ref[...]


def _final_kernel(pa, pb, node_states, u1, bu1, u2, bu2, mx, bt):
    n = pa.shape[0]
    u1a = u1[:, :H]
    u1b = u1[:, H:]
    return pl.pallas_call(
        _final_body,
        grid=(n // bt,),
        in_specs=[
            pl.BlockSpec((bt, H), lambda i: (i, 0)),
            pl.BlockSpec((bt, H), lambda i: (i, 0)),
            pl.BlockSpec((bt, H), lambda i: (i, 0)),
            pl.BlockSpec((2 * H, H), lambda i: (0, 0)),
            pl.BlockSpec((2 * H, H), lambda i: (0, 0)),
            pl.BlockSpec((1, 2 * H), lambda i: (0, 0)),
            pl.BlockSpec((H, 2 * H), lambda i: (0, 0)),
            pl.BlockSpec((1, H), lambda i: (0, 0)),
            pl.BlockSpec(memory_space=pltpu.SMEM),
        ],
        out_specs=pl.BlockSpec((bt, H), lambda i: (i, 0)),
        out_shape=jax.ShapeDtypeStruct((n, H), jnp.float32),
    )(pa, pb, node_states, u1a, u1b, bu1.reshape(1, 2 * H), u2,
      bu2.reshape(1, H), mx.reshape(1))


ARITIES = (1, 2, 3)
MLP_BT = (2000, 1000, 1000)
EXP_BT = (2000, 2000, 2000)


def kernel(node_states, rel_0, rel_1, rel_2,
           W1_0, b1_0, W2_0, b2_0,
           W1_1, b1_1, W2_1, b2_1,
           W1_2, b1_2, W2_2, b2_2,
           U1, bU1, U2, bU2):
    rels = (rel_0, rel_1, rel_2)
    mlps = ((W1_0, b1_0, W2_0, b2_0), (W1_1, b1_1, W2_1, b2_1),
            (W1_2, b1_2, W2_2, b2_2))

    xs = _gather_call(node_states, rel_0, rel_1, rel_2)

    os_ = []
    mxs = []
    for a, x, (w1, b1, w2, b2), bt in zip(ARITIES, xs, mlps, MLP_BT):
        o, mx = _relation_mlp(x.reshape(-1, a * H), w1, b1, w2, b2, bt)
        os_.append(o)
        mxs.append(mx)
    max_offset = jnp.maximum(jnp.maximum(mxs[0], mxs[1]), mxs[2])

    es = [
        _exp_kernel(o, max_offset, bt).reshape(-1, H)
        for o, bt in zip(os_, EXP_BT)
    ]

    zeros = jnp.zeros((N, H), dtype=jnp.float32)
    pa, pb = _scatter_call(es[0], es[1], es[2], rel_0, rel_1, rel_2, zeros)

    return _final_kernel(pa, pb, node_states, U1, bU1, U2, bU2,
                         max_offset, 1000)


# speedup vs baseline: 3.4077x; 1.5631x over previous
import functools

import jax
import jax.numpy as jnp
from jax import lax
from jax.experimental import pallas as pl
from jax.experimental.pallas import tpu as pltpu
from jax.experimental.pallas import tpu_sc as plsc

H = 128
N = 10000
NC = 2
NS = 16
NW = NC * NS

CNT0, CNT1, CNT2 = 300000, 200000, 60000
CH0, CH1, CH2 = 120, 80, 120
CW = 128

_ZROWS = 624
_ZREM = N - _ZROWS * NS


def _worker_id():
    return lax.axis_index("s") * NC + lax.axis_index("c")


def _gather_body(ns_hbm, idx0, idx1, idx2, zeros16, ones16,
                 cnta, cntb, out1, out2,
                 acc16, i0_v, ones_v, ia_v, ra_v, ib_v, rb_v, sem):
    cid = lax.axis_index("c")
    sid = lax.axis_index("s")
    wid = _worker_id()

    row0 = sid * _ZROWS
    pltpu.sync_copy(zeros16.at[pl.ds(row0, _ZROWS)],
                    acc16.at[pl.ds(row0, _ZROWS)])

    @pl.when(sid == NS - 1)
    def _():
        pltpu.sync_copy(zeros16.at[pl.ds(_ZROWS * NS, _ZREM)],
                        acc16.at[pl.ds(_ZROWS * NS, _ZREM)])

    pltpu.sync_copy(ones16, ones_v)
    plsc.subcore_barrier()

    nh = CNT0 // CH0

    def hbody(t, carry):
        j = wid + NW * t

        @pl.when(j < nh)
        def _():
            pltpu.sync_copy(idx0.at[pl.ds(j * CH0, CH0)], i0_v)
            pltpu.sync_copy(ones_v, acc16.at[i0_v], add=True)
        return carry

    lax.fori_loop(0, (nh + NW - 1) // NW, hbody, 0)

    def do_rel(idx_hbm, out_hbm, ch, cnt, idx_v, rows_v):
        nchunks = cnt // ch

        def body(t, carry):
            j = wid + NW * t

            @pl.when(j < nchunks)
            def _():
                base = j * ch
                pltpu.sync_copy(idx_hbm.at[pl.ds(base, ch)], idx_v)
                pltpu.async_copy(ns_hbm.at[idx_v], rows_v, sem).wait()
                pltpu.sync_copy(rows_v, out_hbm.at[pl.ds(base, ch)])
            return carry

        lax.fori_loop(0, (nchunks + NW - 1) // NW, body, 0)

    do_rel(idx1, out1, CH1, CNT1, ib_v, rb_v)
    do_rel(idx2, out2, CH2, CNT2, ia_v, ra_v)

    plsc.subcore_barrier()

    def drain(dst):
        pltpu.sync_copy(acc16.at[pl.ds(row0, _ZROWS)],
                        dst.at[pl.ds(row0, _ZROWS)])

        @pl.when(sid == NS - 1)
        def _():
            pltpu.sync_copy(acc16.at[pl.ds(_ZROWS * NS, _ZREM)],
                            dst.at[pl.ds(_ZROWS * NS, _ZREM)])

    @pl.when(cid == 0)
    def _():
        drain(cnta)

    @pl.when(cid == 1)
    def _():
        drain(cntb)


_gather_call = pl.kernel(
    _gather_body,
    out_type=[jax.ShapeDtypeStruct((N, CW), jnp.float32),
              jax.ShapeDtypeStruct((N, CW), jnp.float32),
              jax.ShapeDtypeStruct((CNT1, H), jnp.float32),
              jax.ShapeDtypeStruct((CNT2, H), jnp.float32)],
    mesh=plsc.VectorSubcoreMesh(core_axis_name="c", subcore_axis_name="s",
                                num_cores=NC, num_subcores=NS),
    scratch_types=[
        pltpu.VMEM_SHARED((N, CW), jnp.float32),
        pltpu.VMEM((CH0,), jnp.int32),
        pltpu.VMEM((CH0, CW), jnp.float32),
        pltpu.VMEM((CH2,), jnp.int32),
        pltpu.VMEM((CH2, H), jnp.float32),
        pltpu.VMEM((CH1,), jnp.int32),
        pltpu.VMEM((CH1, H), jnp.float32),
        pltpu.SemaphoreType.DMA,
    ],
)


def _scatter_body(e1, e2, idx1, idx2, zeros_hbm, outa, outb,
                  acc_sh, ia_v, ra_v, ib_v, rb_v, sem):
    cid = lax.axis_index("c")
    sid = lax.axis_index("s")
    wid = _worker_id()

    row0 = sid * _ZROWS
    pltpu.sync_copy(zeros_hbm.at[pl.ds(row0, _ZROWS)],
                    acc_sh.at[pl.ds(row0, _ZROWS)])

    @pl.when(sid == NS - 1)
    def _():
        pltpu.sync_copy(zeros_hbm.at[pl.ds(_ZROWS * NS, _ZREM)],
                        acc_sh.at[pl.ds(_ZROWS * NS, _ZREM)])

    plsc.subcore_barrier()

    def do_rel(e_hbm, idx_hbm, ch, cnt, idx_v, rows_v):
        nchunks = cnt // ch

        def body(t, carry):
            j = wid + NW * t

            @pl.when(j < nchunks)
            def _():
                base = j * ch
                pltpu.sync_copy(idx_hbm.at[pl.ds(base, ch)], idx_v)
                pltpu.sync_copy(e_hbm.at[pl.ds(base, ch)], rows_v)
                pltpu.sync_copy(rows_v, acc_sh.at[idx_v], add=True)
            return carry

        lax.fori_loop(0, (nchunks + NW - 1) // NW, body, 0)

    do_rel(e1, idx1, CH1, CNT1, ib_v, rb_v)
    do_rel(e2, idx2, CH2, CNT2, ia_v, ra_v)

    plsc.subcore_barrier()

    def drain(dst):
        pltpu.sync_copy(acc_sh.at[pl.ds(row0, _ZROWS)],
                        dst.at[pl.ds(row0, _ZROWS)])

        @pl.when(sid == NS - 1)
        def _():
            pltpu.sync_copy(acc_sh.at[pl.ds(_ZROWS * NS, _ZREM)],
                            dst.at[pl.ds(_ZROWS * NS, _ZREM)])

    @pl.when(cid == 0)
    def _():
        drain(outa)

    @pl.when(cid == 1)
    def _():
        drain(outb)


_scatter_call = pl.kernel(
    _scatter_body,
    out_type=[jax.ShapeDtypeStruct((N, H), jnp.float32),
              jax.ShapeDtypeStruct((N, H), jnp.float32)],
    mesh=plsc.VectorSubcoreMesh(core_axis_name="c", subcore_axis_name="s",
                                num_cores=NC, num_subcores=NS),
    scratch_types=[
        pltpu.VMEM_SHARED((N, H), jnp.float32),
        pltpu.VMEM((CH2,), jnp.int32),
        pltpu.VMEM((CH2, H), jnp.float32),
        pltpu.VMEM((CH1,), jnp.int32),
        pltpu.VMEM((CH1, H), jnp.float32),
        pltpu.SemaphoreType.DMA,
    ],
)


def _mlp_body(x_ref, w1_ref, b1_ref, w2_ref, b2_ref, o_ref, mx_ref):
    x = x_ref[...]
    h = lax.dot_general(x, w1_ref[...], (((1,), (1,)), ((), ())),
                        preferred_element_type=jnp.float32) + b1_ref[...]
    h = jnp.maximum(h, 0.0)
    o = lax.dot_general(h, w2_ref[...], (((1,), (1,)), ((), ())),
                        preferred_element_type=jnp.float32) + b2_ref[...]
    o_ref[...] = o
    bm = jnp.full((1, 1), jnp.max(o), jnp.float32)
    i = pl.program_id(0)

    @pl.when(i == 0)
    def _():
        mx_ref[...] = bm

    @pl.when(i > 0)
    def _():
        mx_ref[...] = jnp.maximum(mx_ref[...], bm)


def _relation_mlp(x, w1, b1, w2, b2, bt):
    t, io = x.shape
    o, mx = pl.pallas_call(
        _mlp_body,
        grid=(t // bt,),
        in_specs=[
            pl.BlockSpec((bt, io), lambda i: (i, 0)),
            pl.BlockSpec((io, io), lambda i: (0, 0)),
            pl.BlockSpec((1, io), lambda i: (0, 0)),
            pl.BlockSpec((io, io), lambda i: (0, 0)),
            pl.BlockSpec((1, io), lambda i: (0, 0)),
        ],
        out_specs=[
            pl.BlockSpec((bt, io), lambda i: (i, 0)),
            pl.BlockSpec((1, 1), lambda i: (0, 0)),
        ],
        out_shape=[
            jax.ShapeDtypeStruct((t, io), jnp.float32),
            jax.ShapeDtypeStruct((1, 1), jnp.float32),
        ],
    )(x, w1, b1.reshape(1, io), w2, b2.reshape(1, io))
    return o, mx[0, 0]


def _masked_max_body(o_ref, ca_ref, cb_ref, mx_ref):
    o = o_ref[...]
    c = ca_ref[...][:, :1] + cb_ref[...][:, :1]
    rm = jnp.max(o, axis=1, keepdims=True)
    masked = jnp.where(c > 0.0, rm, jnp.float32(-jnp.inf))
    bm = jnp.full((1, 1), jnp.max(masked), jnp.float32)
    i = pl.program_id(0)

    @pl.when(i == 0)
    def _():
        mx_ref[...] = bm

    @pl.when(i > 0)
    def _():
        mx_ref[...] = jnp.maximum(mx_ref[...], bm)


def _masked_max(o0, cnta, cntb, bt):
    mx = pl.pallas_call(
        _masked_max_body,
        grid=(N // bt,),
        in_specs=[
            pl.BlockSpec((bt, H), lambda i: (i, 0)),
            pl.BlockSpec((bt, CW), lambda i: (i, 0)),
            pl.BlockSpec((bt, CW), lambda i: (i, 0)),
        ],
        out_specs=pl.BlockSpec((1, 1), lambda i: (0, 0)),
        out_shape=jax.ShapeDtypeStruct((1, 1), jnp.float32),
    )(o0, cnta, cntb)
    return mx[0, 0]


def _exp_body(o_ref, m_ref, e_ref):
    e_ref[...] = jnp.exp(8.0 * (o_ref[...] - m_ref[0]))


def _exp_kernel(o, mx, bt):
    t, io = o.shape
    return pl.pallas_call(
        _exp_body,
        grid=(t // bt,),
        in_specs=[
            pl.BlockSpec((bt, io), lambda i: (i, 0)),
            pl.BlockSpec(memory_space=pltpu.SMEM),
        ],
        out_specs=pl.BlockSpec((bt, io), lambda i: (i, 0)),
        out_shape=jax.ShapeDtypeStruct((t, io), jnp.float32),
    )(o, mx.reshape(1))


def _final_body(pa_ref, pb_ref, o0_ref, ca_ref, cb_ref, ns_ref,
                u1a_ref, u1b_ref, bu1_ref, u2_ref, bu2_ref, m_ref, out_ref):
    c = ca_ref[...][:, :1] + cb_ref[...][:, :1]
    e0 = jnp.exp(8.0 * (o0_ref[...] - m_ref[0]))
    hist = jnp.where(c > 0.0, c * e0, 0.0)
    p = pa_ref[...] + pb_ref[...] + hist + 1e-16
    mm = jnp.log(p) * 0.125 + m_ref[0]
    h = (lax.dot_general(mm, u1a_ref[...], (((1,), (1,)), ((), ())),
                         preferred_element_type=jnp.float32)
         + lax.dot_general(ns_ref[...], u1b_ref[...], (((1,), (1,)), ((), ())),
                           preferred_element_type=jnp.float32)
         + bu1_ref[...])
    h = jnp.maximum(h, 0.0)
    out_ref[...] = lax.dot_general(h, u2_ref[...], (((1,), (1,)), ((), ())),
                                   preferred_element_type=jnp.float32) + bu2_ref[...]


def _final_kernel(pa, pb, o0, cnta, cntb, node_states, u1, bu1, u2, bu2,
                  mx, bt):
    u1a = u1[:, :H]
    u1b = u1[:, H:]
    return pl.pallas_call(
        _final_body,
        grid=(N // bt,),
        in_specs=[
            pl.BlockSpec((bt, H), lambda i: (i, 0)),
            pl.BlockSpec((bt, H), lambda i: (i, 0)),
            pl.BlockSpec((bt, H), lambda i: (i, 0)),
            pl.BlockSpec((bt, CW), lambda i: (i, 0)),
            pl.BlockSpec((bt, CW), lambda i: (i, 0)),
            pl.BlockSpec((bt, H), lambda i: (i, 0)),
            pl.BlockSpec((2 * H, H), lambda i: (0, 0)),
            pl.BlockSpec((2 * H, H), lambda i: (0, 0)),
            pl.BlockSpec((1, 2 * H), lambda i: (0, 0)),
            pl.BlockSpec((H, 2 * H), lambda i: (0, 0)),
            pl.BlockSpec((1, H), lambda i: (0, 0)),
            pl.BlockSpec(memory_space=pltpu.SMEM),
        ],
        out_specs=pl.BlockSpec((bt, H), lambda i: (i, 0)),
        out_shape=jax.ShapeDtypeStruct((N, H), jnp.float32),
    )(pa, pb, o0, cnta, cntb, node_states, u1a, u1b,
      bu1.reshape(1, 2 * H), u2, bu2.reshape(1, H), mx.reshape(1))


def kernel(node_states, rel_0, rel_1, rel_2,
           W1_0, b1_0, W2_0, b2_0,
           W1_1, b1_1, W2_1, b2_1,
           W1_2, b1_2, W2_2, b2_2,
           U1, bU1, U2, bU2):
    zeros16 = jnp.zeros((N, CW), dtype=jnp.float32)
    ones16 = jnp.ones((CH0, CW), dtype=jnp.float32)

    cnta, cntb, x1, x2 = _gather_call(node_states, rel_0, rel_1, rel_2,
                                      zeros16, ones16)

    o0, _ = _relation_mlp(node_states, W1_0, b1_0, W2_0, b2_0, 2000)
    mx0 = _masked_max(o0, cnta, cntb, 2000)

    o1, mx1 = _relation_mlp(x1.reshape(-1, 2 * H), W1_1, b1_1, W2_1, b2_1, 1000)
    o2, mx2 = _relation_mlp(x2.reshape(-1, 3 * H), W1_2, b1_2, W2_2, b2_2, 1000)
    max_offset = jnp.maximum(jnp.maximum(mx0, mx1), mx2)

    e1 = _exp_kernel(o1, max_offset, 2000).reshape(-1, H)
    e2 = _exp_kernel(o2, max_offset, 2000).reshape(-1, H)

    zeros = jnp.zeros((N, H), dtype=jnp.float32)
    pa, pb = _scatter_call(e1, e2, rel_1, rel_2, zeros)

    return _final_kernel(pa, pb, o0, cnta, cntb, node_states,
                         U1, bU1, U2, bU2, max_offset, 1000)
